# Initial kernel scaffold; baseline (speedup 1.0000x reference)
import jax
import jax.numpy as jnp
from jax.experimental import pallas as pl


def kernel(x, edge_index, bn_gamma, bn_beta, bn_mean, bn_var, gcn_W, gcn_b, fc1_W, fc1_b, out_W, out_b):
    raise NotImplementedError("write your pallas kernel here")


# speedup vs baseline: 24.7493x; 24.7493x over previous
import functools

import jax
import jax.numpy as jnp
from jax import lax
from jax.experimental import pallas as pl
from jax.experimental.pallas import tpu as pltpu
from jax.experimental.pallas import tpu_sc as plsc

N = 10000
D = 128
E = 320000
EPS = 1e-5

NC = 2
NS = 16
NW = NC * NS
EPW = E // NW
K = 80
NCH = EPW // K
NPAD = 10240
RPT = NPAD // NS
ZR = 128

RB = 1000
GRID = N // RB


def _sc_degree(dst_r):
    mesh = plsc.VectorSubcoreMesh(core_axis_name="c", subcore_axis_name="s")

    @functools.partial(
        pl.kernel,
        out_type=jax.ShapeDtypeStruct((NC, NPAD), jnp.float32),
        mesh=mesh,
        scratch_types=[
            pltpu.VMEM((NCH, K), jnp.int32),
            pltpu.VMEM((K,), jnp.float32),
            pltpu.VMEM((RPT,), jnp.float32),
            pltpu.VMEM_SHARED((NPAD,), jnp.float32),
        ],
    )
    def deg_kernel(dst_hbm, deg_hbm, idx_v, ones_v, z_v, deg_sh):
        c = lax.axis_index("c")
        s = lax.axis_index("s")
        wid = s * NC + c

        def zfill(i, _):
            z_v[pl.ds(i * 16, 16)] = jnp.zeros((16,), jnp.float32)
            return 0

        lax.fori_loop(0, RPT // 16, zfill, 0)
        pltpu.sync_copy(z_v, deg_sh.at[pl.ds(s * RPT, RPT)])

        def ofill(i, _):
            ones_v[pl.ds(i * 16, 16)] = jnp.ones((16,), jnp.float32)
            return 0

        lax.fori_loop(0, K // 16, ofill, 0)
        pltpu.sync_copy(dst_hbm.at[wid], idx_v)
        plsc.subcore_barrier()

        def body(j, _):
            pltpu.sync_copy(ones_v, deg_sh.at[idx_v.at[j]], add=True)
            return 0

        lax.fori_loop(0, NCH, body, 0)
        plsc.subcore_barrier()
        pltpu.sync_copy(deg_sh.at[pl.ds(s * RPT, RPT)],
                        deg_hbm.at[c, pl.ds(s * RPT, RPT)])

    return deg_kernel(dst_r)


def _sc_scatter(hts, src_r, dst_r):
    mesh = plsc.VectorSubcoreMesh(core_axis_name="c", subcore_axis_name="s")

    @functools.partial(
        pl.kernel,
        out_type=jax.ShapeDtypeStruct((NC, NPAD, D), jnp.float32),
        mesh=mesh,
        scratch_types=[
            pltpu.VMEM((NCH, K), jnp.int32),
            pltpu.VMEM((NCH, K), jnp.int32),
            pltpu.VMEM((K, D), jnp.float32),
            pltpu.VMEM_SHARED((NPAD, D), jnp.float32),
        ],
    )
    def edge_kernel(hts_hbm, src_hbm, dst_hbm, agg_hbm,
                    si_v, di_v, buf_v, acc_sh):
        c = lax.axis_index("c")
        s = lax.axis_index("s")
        wid = s * NC + c

        def zfill(i, _):
            buf_v[i // 8, pl.ds((i % 8) * 16, 16)] = jnp.zeros((16,), jnp.float32)
            return 0

        lax.fori_loop(0, K * (D // 16), zfill, 0)

        def zcopy(b, _):
            pltpu.sync_copy(buf_v, acc_sh.at[pl.ds(s * RPT + b * K, K), :])
            return 0

        lax.fori_loop(0, RPT // K, zcopy, 0)

        pltpu.sync_copy(src_hbm.at[wid], si_v)
        pltpu.sync_copy(dst_hbm.at[wid], di_v)
        plsc.subcore_barrier()

        def body(j, _):
            pltpu.sync_copy(hts_hbm.at[si_v.at[j]], buf_v)
            pltpu.sync_copy(buf_v, acc_sh.at[di_v.at[j]], add=True)
            return 0

        lax.fori_loop(0, NCH, body, 0)
        plsc.subcore_barrier()
        pltpu.sync_copy(acc_sh.at[pl.ds(s * RPT, RPT), :],
                        agg_hbm.at[c, pl.ds(s * RPT, RPT), :])

    return edge_kernel(hts, src_r, dst_r)


def _tc_pre_body(x_r, degp_r, bg_r, bb_r, bm_r, bv_r, gw_r, gb_r, fw_r, fb_r,
                 hts_r, base_r, dinv_r):
    scale = bg_r[...] * lax.rsqrt(bv_r[...] + EPS)
    inp = jnp.maximum((x_r[...] - bm_r[...]) * scale + bb_r[...], 0.0)
    ht = jnp.dot(inp, gw_r[...], preferred_element_type=jnp.float32)
    deg = degp_r[0] + degp_r[1] + 1.0
    dinv = lax.rsqrt(deg)
    hts_r[...] = ht * dinv
    base_r[...] = inp + jnp.dot(inp, fw_r[...],
                                preferred_element_type=jnp.float32) \
        + fb_r[...] + gb_r[...]
    dinv_r[...] = dinv


def _tc_pre(x, degp3, bn_gamma, bn_beta, bn_mean, bn_var,
            gcn_W, gcn_b, fc1_W, fc1_b):
    vec = pl.BlockSpec((1, D), lambda j: (0, 0))
    mat = pl.BlockSpec((D, D), lambda j: (0, 0))
    return pl.pallas_call(
        _tc_pre_body,
        grid=(GRID,),
        in_specs=[
            pl.BlockSpec((RB, D), lambda j: (j, 0)),
            pl.BlockSpec((NC, RB, 1), lambda j: (0, j, 0)),
            vec, vec, vec, vec, mat, vec, mat, vec,
        ],
        out_specs=[
            pl.BlockSpec((RB, D), lambda j: (j, 0)),
            pl.BlockSpec((RB, D), lambda j: (j, 0)),
            pl.BlockSpec((RB, 1), lambda j: (j, 0)),
        ],
        out_shape=[
            jax.ShapeDtypeStruct((N, D), jnp.float32),
            jax.ShapeDtypeStruct((N, D), jnp.float32),
            jax.ShapeDtypeStruct((N, 1), jnp.float32),
        ],
    )(x, degp3, bn_gamma, bn_beta, bn_mean, bn_var, gcn_W, gcn_b, fc1_W, fc1_b)


def _tc_post_body(aggp_r, hts_r, base_r, dinv_r, ow_r, ob_r, fin_r):
    agg = aggp_r[0] + aggp_r[1] + hts_r[...]
    node1 = dinv_r[...] * agg + base_r[...]
    fin_r[...] = jnp.dot(jnp.maximum(node1, 0.0), ow_r[...],
                         preferred_element_type=jnp.float32) + ob_r[...]


def _tc_post(aggp, hts, base, dinv, out_W, out_b):
    return pl.pallas_call(
        _tc_post_body,
        grid=(GRID,),
        in_specs=[
            pl.BlockSpec((NC, RB, D), lambda j: (0, j, 0)),
            pl.BlockSpec((RB, D), lambda j: (j, 0)),
            pl.BlockSpec((RB, D), lambda j: (j, 0)),
            pl.BlockSpec((RB, 1), lambda j: (j, 0)),
            pl.BlockSpec((D, D), lambda j: (0, 0)),
            pl.BlockSpec((1, D), lambda j: (0, 0)),
        ],
        out_specs=pl.BlockSpec((RB, D), lambda j: (j, 0)),
        out_shape=jax.ShapeDtypeStruct((N, D), jnp.float32),
    )(aggp, hts, base, dinv, out_W, out_b)


def kernel(x, edge_index, bn_gamma, bn_beta, bn_mean, bn_var,
           gcn_W, gcn_b, fc1_W, fc1_b, out_W, out_b):
    src_r = edge_index[0].reshape(NW, NCH, K)
    dst_r = edge_index[1].reshape(NW, NCH, K)

    degp = _sc_degree(dst_r)
    degp3 = degp.reshape(NC, NPAD, 1)

    hts, base, dinv = _tc_pre(
        x, degp3,
        bn_gamma.reshape(1, D), bn_beta.reshape(1, D),
        bn_mean.reshape(1, D), bn_var.reshape(1, D),
        gcn_W, gcn_b.reshape(1, D), fc1_W, fc1_b.reshape(1, D))

    aggp = _sc_scatter(hts, src_r, dst_r)

    return _tc_post(aggp, hts, base, dinv, out_W, out_b.reshape(1, D))
